# baseline (speedup 1.0000x reference)
import functools

import jax
import jax.numpy as jnp
from jax.experimental import pallas as pl
from jax.experimental.pallas import tpu as pltpu
from jax.experimental.pallas import tpu_sc as plsc
from jax import lax

B, C_IN, L = 16, 128, 2048
HID, EMB, K = 256, 64, 1024
LH = L // 4
LOUT = L // 4
_NCH = L // 128

_DEF = jax.lax.Precision.DEFAULT


def _shift_right_cols(v):
    lane = jax.lax.broadcasted_iota(jnp.int32, v.shape, 1)
    r = jnp.roll(v, 1, axis=1)
    return jnp.where(lane == 0, 0.0, r)


def _shift_left_cols(v):
    lane = jax.lax.broadcasted_iota(jnp.int32, v.shape, 1)
    r = jnp.roll(v, -1, axis=1)
    return jnp.where(lane == v.shape[1] - 1, 0.0, r)


def _vq_kernel(xa_ref, xb_ref, xc_ref, xd_ref, w1_ref, b1_ref, w2_ref, b2_ref,
               cb_ref, ze_ref, idx_ref, idxh_ref, csq_ref):
    a = xa_ref[0, 0]
    bq = xb_ref[0, 0]
    c = xc_ref[0, 0]
    d = xd_ref[0, 0]

    d_sr = _shift_right_cols(d)
    a_sl = _shift_left_cols(a)

    w1s = w1_ref[...]
    xe_stack = jnp.concatenate([d_sr, a, bq, c], axis=0)
    xo_stack = jnp.concatenate([bq, c, d, a_sl], axis=0)
    bias1 = b1_ref[...]
    z1e = jax.nn.relu(
        jax.lax.dot_general(w1s, xe_stack, (((1,), (0,)), ((), ())),
                            precision=_DEF) + bias1)
    z1o = jax.nn.relu(
        jax.lax.dot_general(w1s, xo_stack, (((1,), (0,)), ((), ())),
                            precision=_DEF) + bias1)

    w2s = w2_ref[...]
    g0 = jax.lax.dot_general(w2s[0], z1o, (((1,), (0,)), ((), ())),
                             precision=_DEF)
    g12 = jax.lax.dot_general(
        jnp.concatenate([w2s[1], w2s[2]], axis=1),
        jnp.concatenate([z1e, z1o], axis=0),
        (((1,), (0,)), ((), ())), precision=_DEF)
    g3 = jax.lax.dot_general(w2s[3], z1e, (((1,), (0,)), ((), ())),
                             precision=_DEF)
    ze = _shift_right_cols(g0) + g12 + _shift_left_cols(g3) + b2_ref[...]
    ze_ref[0] = ze

    cb = cb_ref[...]

    @pl.when(pl.program_id(0) == 0)
    def _():
        csq_ref[...] = jnp.sum(cb * cb, axis=1, keepdims=True)

    csq = csq_ref[...]
    zsq = jnp.sum(ze * ze, axis=0, keepdims=True)
    cross = jax.lax.dot_general(cb, ze, (((1,), (0,)), ((), ())),
                                precision=_DEF)
    d2 = jnp.maximum(csq + zsq - 2.0 * cross, 0.0)

    mins = jnp.min(d2, axis=0, keepdims=True)
    kio = jax.lax.broadcasted_iota(jnp.int32, d2.shape, 0)
    idx = jnp.min(jnp.where(d2 == mins, kio, K), axis=0, keepdims=True)
    idx_ref[0] = idx
    idxh_ref[0] = jax.lax.shift_right_logical(idx, 1)


def _sc_gather_zq(codebook, idx_flat):
    info = plsc.get_sparse_core_info()
    nw = info.num_cores * info.num_subcores
    b_per_w = (B * LOUT) // nw
    mesh = plsc.VectorSubcoreMesh(core_axis_name="c", subcore_axis_name="s")
    table = codebook.reshape(K // 2, 2 * EMB)

    @functools.partial(
        pl.kernel, mesh=mesh,
        out_type=jax.ShapeDtypeStruct((B * LOUT, 2 * EMB), jnp.float32),
        scratch_types=[
            pltpu.VMEM((b_per_w,), jnp.int32),
            pltpu.VMEM((b_per_w, 2 * EMB), jnp.float32),
            pltpu.SemaphoreType.DMA,
        ],
    )
    def k(table_hbm, idx_hbm, out_hbm, idx_v, rows_v, sem):
        wid = lax.axis_index("s") * info.num_cores + lax.axis_index("c")
        base = wid * b_per_w
        pltpu.sync_copy(idx_hbm.at[pl.ds(base, b_per_w)], idx_v)
        pltpu.async_copy(table_hbm.at[idx_v], rows_v, sem).wait()
        pltpu.sync_copy(rows_v, out_hbm.at[pl.ds(base, b_per_w)])

    return k(table, idx_flat)


@functools.partial(jax.jit, static_argnames=())
def kernel(x, W1, b1, W2, b2, codebook):
    xt = jnp.transpose(x.reshape(B, C_IN, LH, 4), (0, 3, 1, 2))
    w1s = jnp.transpose(W1, (0, 2, 1)).reshape(HID, 4 * C_IN)
    w2r = jnp.transpose(W2, (2, 0, 1))
    b1c = b1.reshape(HID, 1)
    b2c = b2.reshape(EMB, 1)

    grid = (B,)
    ze, idx, idxh = pl.pallas_call(
        _vq_kernel,
        grid=grid,
        in_specs=[
            pl.BlockSpec((1, 1, C_IN, LH), lambda i: (i, 0, 0, 0)),
            pl.BlockSpec((1, 1, C_IN, LH), lambda i: (i, 1, 0, 0)),
            pl.BlockSpec((1, 1, C_IN, LH), lambda i: (i, 2, 0, 0)),
            pl.BlockSpec((1, 1, C_IN, LH), lambda i: (i, 3, 0, 0)),
            pl.BlockSpec((HID, 4 * C_IN), lambda i: (0, 0)),
            pl.BlockSpec((HID, 1), lambda i: (0, 0)),
            pl.BlockSpec((4, EMB, HID), lambda i: (0, 0, 0)),
            pl.BlockSpec((EMB, 1), lambda i: (0, 0)),
            pl.BlockSpec((K, EMB), lambda i: (0, 0)),
        ],
        out_specs=[
            pl.BlockSpec((1, EMB, LOUT), lambda i: (i, 0, 0)),
            pl.BlockSpec((1, 1, LOUT), lambda i: (i, 0, 0)),
            pl.BlockSpec((1, 1, LOUT), lambda i: (i, 0, 0)),
        ],
        out_shape=[
            jax.ShapeDtypeStruct((B, EMB, LOUT), jnp.float32),
            jax.ShapeDtypeStruct((B, 1, LOUT), jnp.int32),
            jax.ShapeDtypeStruct((B, 1, LOUT), jnp.int32),
        ],
        scratch_shapes=[pltpu.VMEM((K, 1), jnp.float32)],
    )(xt, xt, xt, xt, w1s, b1c, w2r, b2c, codebook)

    encoding_indices = idx.reshape(B * LOUT)
    pairs = _sc_gather_zq(codebook, idxh.reshape(B * LOUT))
    half = jnp.where((encoding_indices % 2)[:, None] == 1,
                     pairs[:, EMB:], pairs[:, :EMB])
    z_q = half.reshape(B, EMB, LOUT)
    return (z_q, encoding_indices, ze)


# speedup vs baseline: 1.1774x; 1.1774x over previous
import functools

import jax
import jax.numpy as jnp
from jax.experimental import pallas as pl
from jax.experimental.pallas import tpu as pltpu

B, C_IN, L = 16, 128, 2048
HID, EMB, K = 256, 64, 1024
LH = L // 4
LOUT = L // 4
_NCH = L // 128

_DEF = jax.lax.Precision.DEFAULT


def _shift_right_cols(v):
    lane = jax.lax.broadcasted_iota(jnp.int32, v.shape, 1)
    r = jnp.roll(v, 1, axis=1)
    return jnp.where(lane == 0, 0.0, r)


def _shift_left_cols(v):
    lane = jax.lax.broadcasted_iota(jnp.int32, v.shape, 1)
    r = jnp.roll(v, -1, axis=1)
    return jnp.where(lane == v.shape[1] - 1, 0.0, r)


def _vq_kernel(xa_ref, xb_ref, xc_ref, xd_ref, w1_ref, b1_ref, w2_ref, b2_ref,
               cb_ref, ze_ref, idx_ref, zq_ref, csq_ref):
    a = xa_ref[0, 0]
    bq = xb_ref[0, 0]
    c = xc_ref[0, 0]
    d = xd_ref[0, 0]

    d_sr = _shift_right_cols(d)
    a_sl = _shift_left_cols(a)

    w1s = w1_ref[...]
    xe_stack = jnp.concatenate([d_sr, a, bq, c], axis=0)
    xo_stack = jnp.concatenate([bq, c, d, a_sl], axis=0)
    bias1 = b1_ref[...]
    z1e = jax.nn.relu(
        jax.lax.dot_general(w1s, xe_stack, (((1,), (0,)), ((), ())),
                            precision=_DEF) + bias1)
    z1o = jax.nn.relu(
        jax.lax.dot_general(w1s, xo_stack, (((1,), (0,)), ((), ())),
                            precision=_DEF) + bias1)

    w2s = w2_ref[...]
    g0 = jax.lax.dot_general(w2s[0], z1o, (((1,), (0,)), ((), ())),
                             precision=_DEF)
    g12 = jax.lax.dot_general(
        jnp.concatenate([w2s[1], w2s[2]], axis=1),
        jnp.concatenate([z1e, z1o], axis=0),
        (((1,), (0,)), ((), ())), precision=_DEF)
    g3 = jax.lax.dot_general(w2s[3], z1e, (((1,), (0,)), ((), ())),
                             precision=_DEF)
    ze = _shift_right_cols(g0) + g12 + _shift_left_cols(g3) + b2_ref[...]
    ze_ref[0] = ze

    cb = cb_ref[...]

    @pl.when(pl.program_id(0) == 0)
    def _():
        csq_ref[...] = jnp.sum(cb * cb, axis=1, keepdims=True)

    csq = csq_ref[...]
    zsq = jnp.sum(ze * ze, axis=0, keepdims=True)
    cross = jax.lax.dot_general(cb, ze, (((1,), (0,)), ((), ())),
                                precision=_DEF)
    d2 = jnp.maximum(csq + zsq - 2.0 * cross, 0.0)

    mins = jnp.min(d2, axis=0, keepdims=True)
    kio = jax.lax.broadcasted_iota(jnp.int32, d2.shape, 0)
    idx = jnp.min(jnp.where(d2 == mins, kio, K), axis=0, keepdims=True)
    idx_ref[0] = idx

    onehot = (kio == idx).astype(jnp.float32)
    zq = jax.lax.dot_general(onehot, cb, (((0,), (0,)), ((), ())),
                             precision=_DEF)
    zq3 = zq.reshape(EMB, LOUT // EMB, EMB)
    zq_ref[0] = jnp.concatenate([zq3[:, h, :] for h in range(LOUT // EMB)],
                                axis=1)


@functools.partial(jax.jit, static_argnames=())
def kernel(x, W1, b1, W2, b2, codebook):
    xt = jnp.transpose(x.reshape(B, C_IN, LH, 4), (0, 3, 1, 2))
    w1s = jnp.transpose(W1, (0, 2, 1)).reshape(HID, 4 * C_IN)
    w2r = jnp.transpose(W2, (2, 0, 1))
    b1c = b1.reshape(HID, 1)
    b2c = b2.reshape(EMB, 1)

    grid = (B,)
    ze, idx, zq = pl.pallas_call(
        _vq_kernel,
        grid=grid,
        in_specs=[
            pl.BlockSpec((1, 1, C_IN, LH), lambda i: (i, 0, 0, 0)),
            pl.BlockSpec((1, 1, C_IN, LH), lambda i: (i, 1, 0, 0)),
            pl.BlockSpec((1, 1, C_IN, LH), lambda i: (i, 2, 0, 0)),
            pl.BlockSpec((1, 1, C_IN, LH), lambda i: (i, 3, 0, 0)),
            pl.BlockSpec((HID, 4 * C_IN), lambda i: (0, 0)),
            pl.BlockSpec((HID, 1), lambda i: (0, 0)),
            pl.BlockSpec((4, EMB, HID), lambda i: (0, 0, 0)),
            pl.BlockSpec((EMB, 1), lambda i: (0, 0)),
            pl.BlockSpec((K, EMB), lambda i: (0, 0)),
        ],
        out_specs=[
            pl.BlockSpec((1, EMB, LOUT), lambda i: (i, 0, 0)),
            pl.BlockSpec((1, 1, LOUT), lambda i: (i, 0, 0)),
            pl.BlockSpec((1, EMB, LOUT), lambda i: (i, 0, 0)),
        ],
        out_shape=[
            jax.ShapeDtypeStruct((B, EMB, LOUT), jnp.float32),
            jax.ShapeDtypeStruct((B, 1, LOUT), jnp.int32),
            jax.ShapeDtypeStruct((B, EMB, LOUT), jnp.float32),
        ],
        scratch_shapes=[pltpu.VMEM((K, 1), jnp.float32)],
    )(xt, xt, xt, xt, w1s, b1c, w2r, b2c, codebook)

    encoding_indices = idx.reshape(B * LOUT)
    return (zq, encoding_indices, ze)
